# Initial kernel scaffold; baseline (speedup 1.0000x reference)
import jax
import jax.numpy as jnp
from jax.experimental import pallas as pl


def kernel(x, edge_index, W1, b1, W2, b2, W3, b3, W4, b4):
    raise NotImplementedError("write your pallas kernel here")


# speedup vs baseline: 20.4069x; 20.4069x over previous
import functools

import jax
import jax.numpy as jnp
from jax import lax
from jax.experimental import pallas as pl
from jax.experimental.pallas import tpu as pltpu
from jax.experimental.pallas import tpu_sc as plsc

_BLK = 128
_K = 8
_NTILE = 16
_NSC = 2
_BN = 2000


def _tile_rows(n):
    ra = ((n // _NTILE) + 7) // 8 * 8
    rl = n - (_NTILE - 1) * ra
    assert 0 < rl <= ra
    return ra, rl


def _zero_acc(sid, zeros, acc, ra, rl):
    base = sid * ra

    @pl.when(sid < _NTILE - 1)
    def _():
        pltpu.sync_copy(zeros, acc.at[pl.ds(base, ra)])

    @pl.when(sid == _NTILE - 1)
    def _():
        pltpu.sync_copy(zeros.at[pl.ds(0, rl)], acc.at[pl.ds(base, rl)])


def _flush_acc(cid, sid, acc, out, ra, rl):
    base = sid * ra

    @pl.when(sid < _NTILE - 1)
    def _():
        pltpu.sync_copy(acc.at[pl.ds(base, ra)], out.at[cid].at[pl.ds(base, ra)])

    @pl.when(sid == _NTILE - 1)
    def _():
        pltpu.sync_copy(acc.at[pl.ds(base, rl)], out.at[cid].at[pl.ds(base, rl)])


@functools.lru_cache(maxsize=None)
def _make_prop(n, nb, width):
    ra, rl = _tile_rows(n)
    nw = _NSC * _NTILE
    nsb = nb // _K
    tail = nb - nsb * _K
    mesh = plsc.VectorSubcoreMesh(core_axis_name="c", subcore_axis_name="s",
                                  num_cores=_NSC, num_subcores=_NTILE)

    @functools.partial(
        pl.kernel,
        out_type=jax.ShapeDtypeStruct((_NSC, n, width), jnp.float32),
        mesh=mesh,
        compiler_params=pltpu.CompilerParams(use_tc_tiling_on_sc=False),
        scratch_types=[
            pltpu.VMEM((_K, _BLK), jnp.int32),
            pltpu.VMEM((_K, _BLK), jnp.int32),
            pltpu.VMEM((_K, _BLK, width), jnp.float32),
            pltpu.VMEM_SHARED((n, width), jnp.float32),
            pltpu.SemaphoreType.DMA,
            pltpu.SemaphoreType.DMA,
        ],
    )
    def k(table, srcb, dstb, zeros, out, sidx, didx, rows, acc, gsem, ssem):
        cid = lax.axis_index("c")
        sid = lax.axis_index("s")
        _zero_acc(sid, zeros, acc, ra, rl)
        plsc.subcore_barrier()
        w = sid * _NSC + cid
        cnt = (nsb - w + nw - 1) // nw

        def run_blocks(blk0, m):
            pltpu.sync_copy(srcb.at[pl.ds(blk0, m)], sidx.at[pl.ds(0, m)])
            pltpu.sync_copy(dstb.at[pl.ds(blk0, m)], didx.at[pl.ds(0, m)])
            for j in range(m):
                pltpu.async_copy(table.at[sidx.at[j]], rows.at[j], gsem)
            for j in range(m):
                pltpu.make_async_copy(table.at[sidx.at[j]], rows.at[j], gsem).wait()
            for j in range(m):
                pltpu.async_copy(rows.at[j], acc.at[didx.at[j]], ssem, add=True)
            for j in range(m):
                pltpu.make_async_copy(rows.at[j], acc.at[didx.at[j]], ssem).wait()

        def body(i, carry):
            run_blocks((w + i * nw) * _K, _K)
            return carry

        lax.fori_loop(0, cnt, body, 0)
        if tail:
            @pl.when(w == 0)
            def _():
                run_blocks(nsb * _K, tail)
        plsc.subcore_barrier()
        _flush_acc(cid, sid, acc, out, ra, rl)

    return k


@functools.lru_cache(maxsize=None)
def _make_deg(n, nb):
    width = 16
    ra, rl = _tile_rows(n)
    nw = _NSC * _NTILE
    nsb = nb // _K
    tail = nb - nsb * _K
    mesh = plsc.VectorSubcoreMesh(core_axis_name="c", subcore_axis_name="s",
                                  num_cores=_NSC, num_subcores=_NTILE)

    @functools.partial(
        pl.kernel,
        out_type=jax.ShapeDtypeStruct((_NSC, n, width), jnp.float32),
        mesh=mesh,
        compiler_params=pltpu.CompilerParams(use_tc_tiling_on_sc=False),
        scratch_types=[
            pltpu.VMEM((_K, _BLK), jnp.int32),
            pltpu.VMEM((_BLK, width), jnp.float32),
            pltpu.VMEM_SHARED((n, width), jnp.float32),
            pltpu.SemaphoreType.DMA,
        ],
    )
    def k(dstb, ones, zeros, out, didx, ones_v, acc, ssem):
        cid = lax.axis_index("c")
        sid = lax.axis_index("s")
        pltpu.sync_copy(ones, ones_v)
        _zero_acc(sid, zeros, acc, ra, rl)
        plsc.subcore_barrier()
        w = sid * _NSC + cid
        cnt = (nsb - w + nw - 1) // nw

        def run_blocks(blk0, m):
            pltpu.sync_copy(dstb.at[pl.ds(blk0, m)], didx.at[pl.ds(0, m)])
            for j in range(m):
                pltpu.async_copy(ones_v, acc.at[didx.at[j]], ssem, add=True)
            for j in range(m):
                pltpu.make_async_copy(ones_v, acc.at[didx.at[j]], ssem).wait()

        def body(i, carry):
            run_blocks((w + i * nw) * _K, _K)
            return carry

        lax.fori_loop(0, cnt, body, 0)
        if tail:
            @pl.when(w == 0)
            def _():
                run_blocks(nsb * _K, tail)
        plsc.subcore_barrier()
        _flush_acc(cid, sid, acc, out, ra, rl)

    return k


def _rowcall(body, n, out_widths, blocked, full):
    in_specs = (
        [pl.BlockSpec((_BN, a.shape[1]), lambda i: (i, 0)) for a in blocked]
        + [pl.BlockSpec(a.shape, lambda i, _nd=a.ndim: (0,) * _nd) for a in full]
    )
    out = pl.pallas_call(
        body,
        grid=(n // _BN,),
        in_specs=in_specs,
        out_specs=[pl.BlockSpec((_BN, w), lambda i: (i, 0)) for w in out_widths],
        out_shape=[jax.ShapeDtypeStruct((n, w), jnp.float32) for w in out_widths],
    )(*blocked, *full)
    return out


def _b1(da, db, xr, dinv_o, t1c0_o, t1c1_o):
    deg = da[...][:, :1] + db[...][:, :1] + 1.0
    dinv = lax.rsqrt(deg)
    dinv_o[...] = dinv
    xv = xr[...]
    t1c0_o[...] = xv[:, :16] * dinv
    pad = jnp.zeros((xv.shape[0], 14), jnp.float32)
    t1c1_o[...] = jnp.concatenate([xv[:, 16:18] * dinv, pad], axis=1)


def _b2(s0a, s0b, t0, s1a, s1b, t1, dv, w1, bb1, w2, t2c0_o, t2c1_o):
    dinv = dv[...]
    u0 = (s0a[...] + s0b[...] + t0[...]) * dinv
    u1 = ((s1a[...] + s1b[...] + t1[...]) * dinv)[:, :2]
    out1 = jnp.concatenate([u0, u1], axis=1)
    h1 = jnp.maximum(
        jnp.dot(out1, w1[...], preferred_element_type=jnp.float32) + bb1[...], 0.0)
    g2 = jnp.dot(h1, w2[...], preferred_element_type=jnp.float32)
    t2c0_o[...] = g2[:, :16] * dinv
    t2c1_o[...] = g2[:, 16:] * dinv


def _b3(s0a, s0b, t0, s1a, s1b, t1, dv, bb2, w3, t3_o):
    dinv = dv[...]
    u0 = (s0a[...] + s0b[...] + t0[...]) * dinv
    u1 = (s1a[...] + s1b[...] + t1[...]) * dinv
    out2 = jnp.concatenate([u0, u1], axis=1)
    h2 = jnp.maximum(out2 + bb2[...], 0.0)
    g3 = jnp.dot(h2, w3[...], preferred_element_type=jnp.float32)
    t3_o[...] = g3 * dinv


def _b4(sa, sb, t3, dv, bb3, w4, t4_o):
    dinv = dv[...]
    out3 = (sa[...] + sb[...] + t3[...]) * dinv
    h3 = jnp.maximum(out3 + bb3[...], 0.0)
    g4 = jnp.dot(h3, w4[...], preferred_element_type=jnp.float32)
    pad = jnp.zeros((g4.shape[0], 14), jnp.float32)
    t4_o[...] = jnp.concatenate([g4 * dinv, pad], axis=1)


def _b5(sa, sb, t4, dv, bb4, y_o):
    v = ((sa[...] + sb[...] + t4[...]) * dv[...])[:, :2] + bb4[...]
    m = jnp.max(v, axis=1, keepdims=True)
    z = v - m
    lse = jnp.log(jnp.sum(jnp.exp(z), axis=1, keepdims=True))
    y_o[...] = z - lse


def kernel(x, edge_index, W1, b1, W2, b2, W3, b3, W4, b4):
    n = x.shape[0]
    e = edge_index.shape[1]
    assert e % _BLK == 0 and n % _BN == 0
    nb = e // _BLK
    ra, _ = _tile_rows(n)

    src2 = edge_index[0].reshape(nb, _BLK)
    dst2 = edge_index[1].reshape(nb, _BLK)
    zeros16 = jnp.zeros((ra, 16), jnp.float32)
    ones16 = jnp.zeros((_BLK, 16), jnp.float32).at[:, 0].set(1.0)
    bb1 = b1.reshape(1, -1)
    bb2 = b2.reshape(1, -1)
    bb3 = b3.reshape(1, -1)
    bb4 = b4.reshape(1, -1)

    prop16 = _make_prop(n, nb, 16)

    degp = _make_deg(n, nb)(dst2, ones16, zeros16)
    dinv, t1c0, t1c1 = _rowcall(_b1, n, [1, 16, 16], [degp[0], degp[1], x], [])

    s10 = prop16(t1c0, src2, dst2, zeros16)
    s11 = prop16(t1c1, src2, dst2, zeros16)
    t2c0, t2c1 = _rowcall(
        _b2, n, [16, 16],
        [s10[0], s10[1], t1c0, s11[0], s11[1], t1c1, dinv], [W1, bb1, W2])

    s20 = prop16(t2c0, src2, dst2, zeros16)
    s21 = prop16(t2c1, src2, dst2, zeros16)
    (t3,) = _rowcall(
        _b3, n, [16],
        [s20[0], s20[1], t2c0, s21[0], s21[1], t2c1, dinv], [bb2, W3])

    s3 = prop16(t3, src2, dst2, zeros16)
    (t4,) = _rowcall(_b4, n, [16], [s3[0], s3[1], t3, dinv], [bb3, W4])

    s4 = prop16(t4, src2, dst2, zeros16)
    (y,) = _rowcall(_b5, n, [2], [s4[0], s4[1], t4, dinv], [bb4])
    return y


# speedup vs baseline: 21.4927x; 1.0532x over previous
import functools

import jax
import jax.numpy as jnp
from jax import lax
from jax.experimental import pallas as pl
from jax.experimental.pallas import tpu as pltpu
from jax.experimental.pallas import tpu_sc as plsc

_BLK = 64
_K = 8
_NTILE = 16
_NSC = 2
_BN = 2000


def _tile_rows(n):
    ra = ((n // _NTILE) + 7) // 8 * 8
    rl = n - (_NTILE - 1) * ra
    assert 0 < rl <= ra
    return ra, rl


def _zero_acc(sid, zeros, acc, ra, rl):
    base = sid * ra

    @pl.when(sid < _NTILE - 1)
    def _():
        pltpu.sync_copy(zeros, acc.at[pl.ds(base, ra)])

    @pl.when(sid == _NTILE - 1)
    def _():
        pltpu.sync_copy(zeros.at[pl.ds(0, rl)], acc.at[pl.ds(base, rl)])


def _flush_acc(cid, sid, acc, out, ra, rl):
    base = sid * ra

    @pl.when(sid < _NTILE - 1)
    def _():
        pltpu.sync_copy(acc.at[pl.ds(base, ra)], out.at[cid].at[pl.ds(base, ra)])

    @pl.when(sid == _NTILE - 1)
    def _():
        pltpu.sync_copy(acc.at[pl.ds(base, rl)], out.at[cid].at[pl.ds(base, rl)])


@functools.lru_cache(maxsize=None)
def _make_prop(n, nb, width):
    ra, rl = _tile_rows(n)
    nw = _NSC * _NTILE
    nsb = nb // _K
    tail = nb - nsb * _K
    mesh = plsc.VectorSubcoreMesh(core_axis_name="c", subcore_axis_name="s",
                                  num_cores=_NSC, num_subcores=_NTILE)

    @functools.partial(
        pl.kernel,
        out_type=jax.ShapeDtypeStruct((_NSC, n, width), jnp.float32),
        mesh=mesh,
        compiler_params=pltpu.CompilerParams(use_tc_tiling_on_sc=False),
        scratch_types=[
            pltpu.VMEM((2, _K, _BLK), jnp.int32),
            pltpu.VMEM((2, _K, _BLK), jnp.int32),
            pltpu.VMEM((2, _K, _BLK, width), jnp.float32),
            pltpu.VMEM_SHARED((n, width), jnp.float32),
            pltpu.SemaphoreType.DMA,
            pltpu.SemaphoreType.DMA,
        ],
    )
    def k(table, srcb, dstb, zeros, out, sidx, didx, rows, acc, gsem, ssem):
        cid = lax.axis_index("c")
        sid = lax.axis_index("s")
        _zero_acc(sid, zeros, acc, ra, rl)
        plsc.subcore_barrier()
        w = sid * _NSC + cid
        cnt = (nsb - w + nw - 1) // nw

        def load_and_fire(b, i):
            blk0 = (w + i * nw) * _K
            pltpu.sync_copy(srcb.at[pl.ds(blk0, _K)], sidx.at[b])
            pltpu.sync_copy(dstb.at[pl.ds(blk0, _K)], didx.at[b])
            for j in range(_K):
                pltpu.async_copy(table.at[sidx.at[b].at[j]], rows.at[b].at[j], gsem)

        def drain_gathers(b):
            for j in range(_K):
                pltpu.make_async_copy(
                    table.at[sidx.at[b].at[j]], rows.at[b].at[j], gsem).wait()

        def fire_scatters(b):
            for j in range(_K):
                pltpu.async_copy(rows.at[b].at[j], acc.at[didx.at[b].at[j]],
                                 ssem, add=True)

        def drain_scatters():
            for j in range(_K):
                pltpu.make_async_copy(rows.at[0].at[j], acc.at[didx.at[0].at[j]],
                                      ssem).wait()

        def step(i, bx, by):
            @pl.when(i + 1 < cnt)
            def _():
                @pl.when(i >= 1)
                def _():
                    drain_scatters()
                load_and_fire(by, i + 1)
            drain_gathers(bx)
            fire_scatters(bx)

        @pl.when(cnt > 0)
        def _():
            load_and_fire(0, 0)

        def pair(p, carry):
            step(2 * p, 0, 1)

            @pl.when(2 * p + 1 < cnt)
            def _():
                step(2 * p + 1, 1, 0)
            return carry

        lax.fori_loop(0, (cnt + 1) // 2, pair, 0)
        @pl.when(cnt > 1)
        def _():
            drain_scatters()
        @pl.when(cnt > 0)
        def _():
            drain_scatters()

        if tail:
            @pl.when(w == 0)
            def _():
                blk0 = nsb * _K
                pltpu.sync_copy(srcb.at[pl.ds(blk0, tail)],
                                sidx.at[0].at[pl.ds(0, tail)])
                pltpu.sync_copy(dstb.at[pl.ds(blk0, tail)],
                                didx.at[0].at[pl.ds(0, tail)])
                for j in range(tail):
                    pltpu.async_copy(table.at[sidx.at[0].at[j]],
                                     rows.at[0].at[j], gsem)
                for j in range(tail):
                    pltpu.make_async_copy(table.at[sidx.at[0].at[j]],
                                          rows.at[0].at[j], gsem).wait()
                for j in range(tail):
                    pltpu.async_copy(rows.at[0].at[j], acc.at[didx.at[0].at[j]],
                                     ssem, add=True)
                for j in range(tail):
                    pltpu.make_async_copy(rows.at[0].at[j],
                                          acc.at[didx.at[0].at[j]], ssem).wait()
        plsc.subcore_barrier()
        _flush_acc(cid, sid, acc, out, ra, rl)

    return k


@functools.lru_cache(maxsize=None)
def _make_deg(n, nb):
    width = 16
    ra, rl = _tile_rows(n)
    nw = _NSC * _NTILE
    nsb = nb // _K
    tail = nb - nsb * _K
    mesh = plsc.VectorSubcoreMesh(core_axis_name="c", subcore_axis_name="s",
                                  num_cores=_NSC, num_subcores=_NTILE)

    @functools.partial(
        pl.kernel,
        out_type=jax.ShapeDtypeStruct((_NSC, n, width), jnp.float32),
        mesh=mesh,
        compiler_params=pltpu.CompilerParams(use_tc_tiling_on_sc=False),
        scratch_types=[
            pltpu.VMEM((_K, _BLK), jnp.int32),
            pltpu.VMEM((_BLK, width), jnp.float32),
            pltpu.VMEM_SHARED((n, width), jnp.float32),
            pltpu.SemaphoreType.DMA,
        ],
    )
    def k(dstb, ones, zeros, out, didx, ones_v, acc, ssem):
        cid = lax.axis_index("c")
        sid = lax.axis_index("s")
        pltpu.sync_copy(ones, ones_v)
        _zero_acc(sid, zeros, acc, ra, rl)
        plsc.subcore_barrier()
        w = sid * _NSC + cid
        cnt = (nsb - w + nw - 1) // nw

        def run_blocks(blk0, m):
            pltpu.sync_copy(dstb.at[pl.ds(blk0, m)], didx.at[pl.ds(0, m)])
            for j in range(m):
                pltpu.async_copy(ones_v, acc.at[didx.at[j]], ssem, add=True)
            for j in range(m):
                pltpu.make_async_copy(ones_v, acc.at[didx.at[j]], ssem).wait()

        def body(i, carry):
            run_blocks((w + i * nw) * _K, _K)
            return carry

        lax.fori_loop(0, cnt, body, 0)
        if tail:
            @pl.when(w == 0)
            def _():
                run_blocks(nsb * _K, tail)
        plsc.subcore_barrier()
        _flush_acc(cid, sid, acc, out, ra, rl)

    return k


def _rowcall(body, n, out_widths, blocked, full):
    in_specs = (
        [pl.BlockSpec((_BN, a.shape[1]), lambda i: (i, 0)) for a in blocked]
        + [pl.BlockSpec(a.shape, lambda i, _nd=a.ndim: (0,) * _nd) for a in full]
    )
    out = pl.pallas_call(
        body,
        grid=(n // _BN,),
        in_specs=in_specs,
        out_specs=[pl.BlockSpec((_BN, w), lambda i: (i, 0)) for w in out_widths],
        out_shape=[jax.ShapeDtypeStruct((n, w), jnp.float32) for w in out_widths],
    )(*blocked, *full)
    return out


def _b1(da, db, xr, dinv_o, t1c0_o, t1c1_o):
    deg = da[...][:, :1] + db[...][:, :1] + 1.0
    dinv = lax.rsqrt(deg)
    dinv_o[...] = dinv
    xv = xr[...]
    t1c0_o[...] = xv[:, :16] * dinv
    pad = jnp.zeros((xv.shape[0], 14), jnp.float32)
    t1c1_o[...] = jnp.concatenate([xv[:, 16:18] * dinv, pad], axis=1)


def _b2(s0a, s0b, t0, s1a, s1b, t1, dv, w1, bb1, w2, t2c0_o, t2c1_o):
    dinv = dv[...]
    u0 = (s0a[...] + s0b[...] + t0[...]) * dinv
    u1 = ((s1a[...] + s1b[...] + t1[...]) * dinv)[:, :2]
    out1 = jnp.concatenate([u0, u1], axis=1)
    h1 = jnp.maximum(
        jnp.dot(out1, w1[...], preferred_element_type=jnp.float32) + bb1[...], 0.0)
    g2 = jnp.dot(h1, w2[...], preferred_element_type=jnp.float32)
    t2c0_o[...] = g2[:, :16] * dinv
    t2c1_o[...] = g2[:, 16:] * dinv


def _b3(s0a, s0b, t0, s1a, s1b, t1, dv, bb2, w3, t3_o):
    dinv = dv[...]
    u0 = (s0a[...] + s0b[...] + t0[...]) * dinv
    u1 = (s1a[...] + s1b[...] + t1[...]) * dinv
    out2 = jnp.concatenate([u0, u1], axis=1)
    h2 = jnp.maximum(out2 + bb2[...], 0.0)
    g3 = jnp.dot(h2, w3[...], preferred_element_type=jnp.float32)
    t3_o[...] = g3 * dinv


def _b4(sa, sb, t3, dv, bb3, w4, t4_o):
    dinv = dv[...]
    out3 = (sa[...] + sb[...] + t3[...]) * dinv
    h3 = jnp.maximum(out3 + bb3[...], 0.0)
    g4 = jnp.dot(h3, w4[...], preferred_element_type=jnp.float32)
    pad = jnp.zeros((g4.shape[0], 14), jnp.float32)
    t4_o[...] = jnp.concatenate([g4 * dinv, pad], axis=1)


def _b5(sa, sb, t4, dv, bb4, y_o):
    v = ((sa[...] + sb[...] + t4[...]) * dv[...])[:, :2] + bb4[...]
    m = jnp.max(v, axis=1, keepdims=True)
    z = v - m
    lse = jnp.log(jnp.sum(jnp.exp(z), axis=1, keepdims=True))
    y_o[...] = z - lse


def kernel(x, edge_index, W1, b1, W2, b2, W3, b3, W4, b4):
    n = x.shape[0]
    e = edge_index.shape[1]
    assert e % _BLK == 0 and n % _BN == 0
    nb = e // _BLK
    ra, _ = _tile_rows(n)

    src2 = edge_index[0].reshape(nb, _BLK)
    dst2 = edge_index[1].reshape(nb, _BLK)
    zeros16 = jnp.zeros((ra, 16), jnp.float32)
    ones16 = jnp.zeros((_BLK, 16), jnp.float32).at[:, 0].set(1.0)
    bb1 = b1.reshape(1, -1)
    bb2 = b2.reshape(1, -1)
    bb3 = b3.reshape(1, -1)
    bb4 = b4.reshape(1, -1)

    prop16 = _make_prop(n, nb, 16)

    degp = _make_deg(n, nb)(dst2, ones16, zeros16)
    dinv, t1c0, t1c1 = _rowcall(_b1, n, [1, 16, 16], [degp[0], degp[1], x], [])

    s10 = prop16(t1c0, src2, dst2, zeros16)
    s11 = prop16(t1c1, src2, dst2, zeros16)
    t2c0, t2c1 = _rowcall(
        _b2, n, [16, 16],
        [s10[0], s10[1], t1c0, s11[0], s11[1], t1c1, dinv], [W1, bb1, W2])

    s20 = prop16(t2c0, src2, dst2, zeros16)
    s21 = prop16(t2c1, src2, dst2, zeros16)
    (t3,) = _rowcall(
        _b3, n, [16],
        [s20[0], s20[1], t2c0, s21[0], s21[1], t2c1, dinv], [bb2, W3])

    s3 = prop16(t3, src2, dst2, zeros16)
    (t4,) = _rowcall(_b4, n, [16], [s3[0], s3[1], t3, dinv], [bb3, W4])

    s4 = prop16(t4, src2, dst2, zeros16)
    (y,) = _rowcall(_b5, n, [2], [s4[0], s4[1], t4, dinv], [bb4])
    return y


# speedup vs baseline: 22.1560x; 1.0309x over previous
import functools

import jax
import jax.numpy as jnp
from jax import lax
from jax.experimental import pallas as pl
from jax.experimental.pallas import tpu as pltpu
from jax.experimental.pallas import tpu_sc as plsc

_BLK = 64
_K = 8
_NTILE = 16
_NSC = 2
_BN = 2000


def _tile_rows(n):
    ra = ((n // _NTILE) + 7) // 8 * 8
    rl = n - (_NTILE - 1) * ra
    assert 0 < rl <= ra
    return ra, rl


def _zero_acc_start(sid, zeros, acc, ra, rl, sem):
    base = sid * ra

    @pl.when(sid < _NTILE - 1)
    def _():
        pltpu.async_copy(zeros, acc.at[pl.ds(base, ra)], sem)

    @pl.when(sid == _NTILE - 1)
    def _():
        pltpu.async_copy(zeros.at[pl.ds(0, rl)], acc.at[pl.ds(base, rl)], sem)


def _zero_acc_wait(sid, zeros, acc, ra, rl, sem):
    base = sid * ra

    @pl.when(sid < _NTILE - 1)
    def _():
        pltpu.make_async_copy(zeros, acc.at[pl.ds(base, ra)], sem).wait()

    @pl.when(sid == _NTILE - 1)
    def _():
        pltpu.make_async_copy(zeros.at[pl.ds(0, rl)],
                              acc.at[pl.ds(base, rl)], sem).wait()


def _flush_acc(cid, sid, acc, out, ra, rl):
    base = sid * ra

    @pl.when(sid < _NTILE - 1)
    def _():
        pltpu.sync_copy(acc.at[pl.ds(base, ra)], out.at[cid].at[pl.ds(base, ra)])

    @pl.when(sid == _NTILE - 1)
    def _():
        pltpu.sync_copy(acc.at[pl.ds(base, rl)], out.at[cid].at[pl.ds(base, rl)])


@functools.lru_cache(maxsize=None)
def _make_prop(n, nb, width, chunked=False):
    ra, rl = _tile_rows(n)
    nw = _NTILE if chunked else _NSC * _NTILE
    nsb = nb // _K
    tail = nb - nsb * _K
    mesh = plsc.VectorSubcoreMesh(core_axis_name="c", subcore_axis_name="s",
                                  num_cores=_NSC, num_subcores=_NTILE)
    @functools.partial(
        pl.kernel,
        out_type=jax.ShapeDtypeStruct((_NSC, n, width), jnp.float32),
        mesh=mesh,
        compiler_params=pltpu.CompilerParams(use_tc_tiling_on_sc=False),
        scratch_types=[
            pltpu.VMEM((2, _K, _BLK), jnp.int32),
            pltpu.VMEM((2, _K, _BLK), jnp.int32),
            pltpu.VMEM((2, _K, _BLK, width), jnp.float32),
            pltpu.VMEM_SHARED((n, width), jnp.float32),
            pltpu.SemaphoreType.DMA,
            pltpu.SemaphoreType.DMA,
            pltpu.SemaphoreType.DMA,
        ],
    )
    def k(table_in, srcb, dstb, zeros, out, sidx, didx, rows, acc,
          gsem, ssem, zsem):
        cid = lax.axis_index("c")
        sid = lax.axis_index("s")
        table = table_in.at[cid] if chunked else table_in
        _zero_acc_start(sid, zeros, acc, ra, rl, zsem)
        w = sid if chunked else sid * _NSC + cid
        cnt = (nsb - w + nw - 1) // nw

        def load_and_fire(b, i):
            blk0 = (w + i * nw) * _K
            pltpu.sync_copy(srcb.at[pl.ds(blk0, _K)], sidx.at[b])
            pltpu.sync_copy(dstb.at[pl.ds(blk0, _K)], didx.at[b])
            for j in range(_K):
                pltpu.async_copy(table.at[sidx.at[b].at[j]], rows.at[b].at[j], gsem)

        def drain_gathers(b):
            for j in range(_K):
                pltpu.make_async_copy(
                    table.at[sidx.at[b].at[j]], rows.at[b].at[j], gsem).wait()

        def fire_scatters(b):
            for j in range(_K):
                pltpu.async_copy(rows.at[b].at[j], acc.at[didx.at[b].at[j]],
                                 ssem, add=True)

        def drain_scatters():
            for j in range(_K):
                pltpu.make_async_copy(rows.at[0].at[j], acc.at[didx.at[0].at[j]],
                                      ssem).wait()

        def step(i, bx, by):
            @pl.when(i + 1 < cnt)
            def _():
                @pl.when(i >= 1)
                def _():
                    drain_scatters()
                load_and_fire(by, i + 1)
            drain_gathers(bx)
            fire_scatters(bx)

        @pl.when(cnt > 0)
        def _():
            load_and_fire(0, 0)

        _zero_acc_wait(sid, zeros, acc, ra, rl, zsem)
        plsc.subcore_barrier()

        def pair(p, carry):
            step(2 * p, 0, 1)

            @pl.when(2 * p + 1 < cnt)
            def _():
                step(2 * p + 1, 1, 0)
            return carry

        lax.fori_loop(0, (cnt + 1) // 2, pair, 0)
        @pl.when(cnt > 1)
        def _():
            drain_scatters()
        @pl.when(cnt > 0)
        def _():
            drain_scatters()

        if tail:
            @pl.when(w == 0)
            def _():
                blk0 = nsb * _K
                pltpu.sync_copy(srcb.at[pl.ds(blk0, tail)],
                                sidx.at[0].at[pl.ds(0, tail)])
                pltpu.sync_copy(dstb.at[pl.ds(blk0, tail)],
                                didx.at[0].at[pl.ds(0, tail)])
                for j in range(tail):
                    pltpu.async_copy(table.at[sidx.at[0].at[j]],
                                     rows.at[0].at[j], gsem)
                for j in range(tail):
                    pltpu.make_async_copy(table.at[sidx.at[0].at[j]],
                                          rows.at[0].at[j], gsem).wait()
                for j in range(tail):
                    pltpu.async_copy(rows.at[0].at[j], acc.at[didx.at[0].at[j]],
                                     ssem, add=True)
                for j in range(tail):
                    pltpu.make_async_copy(rows.at[0].at[j],
                                          acc.at[didx.at[0].at[j]], ssem).wait()
        plsc.subcore_barrier()
        _flush_acc(cid, sid, acc, out, ra, rl)

    return k


@functools.lru_cache(maxsize=None)
def _make_deg(n, nb):
    width = 16
    ra, rl = _tile_rows(n)
    nw = _NSC * _NTILE
    nsb = nb // _K
    tail = nb - nsb * _K
    mesh = plsc.VectorSubcoreMesh(core_axis_name="c", subcore_axis_name="s",
                                  num_cores=_NSC, num_subcores=_NTILE)

    @functools.partial(
        pl.kernel,
        out_type=jax.ShapeDtypeStruct((_NSC, n, width), jnp.float32),
        mesh=mesh,
        compiler_params=pltpu.CompilerParams(use_tc_tiling_on_sc=False),
        scratch_types=[
            pltpu.VMEM((_K, _BLK), jnp.int32),
            pltpu.VMEM((_BLK, width), jnp.float32),
            pltpu.VMEM_SHARED((n, width), jnp.float32),
            pltpu.SemaphoreType.DMA,
            pltpu.SemaphoreType.DMA,
        ],
    )
    def k(dstb, ones, zeros, out, didx, ones_v, acc, ssem, zsem):
        cid = lax.axis_index("c")
        sid = lax.axis_index("s")
        _zero_acc_start(sid, zeros, acc, ra, rl, zsem)
        pltpu.sync_copy(ones, ones_v)
        _zero_acc_wait(sid, zeros, acc, ra, rl, zsem)
        plsc.subcore_barrier()
        w = sid * _NSC + cid
        cnt = (nsb - w + nw - 1) // nw

        def run_blocks(blk0, m):
            pltpu.sync_copy(dstb.at[pl.ds(blk0, m)], didx.at[pl.ds(0, m)])
            for j in range(m):
                pltpu.async_copy(ones_v, acc.at[didx.at[j]], ssem, add=True)
            for j in range(m):
                pltpu.make_async_copy(ones_v, acc.at[didx.at[j]], ssem).wait()

        def body(i, carry):
            run_blocks((w + i * nw) * _K, _K)
            return carry

        lax.fori_loop(0, cnt, body, 0)
        if tail:
            @pl.when(w == 0)
            def _():
                run_blocks(nsb * _K, tail)
        plsc.subcore_barrier()
        _flush_acc(cid, sid, acc, out, ra, rl)

    return k


def _rowcall(body, n, out_widths, blocked, full):
    in_specs = (
        [pl.BlockSpec((_BN, a.shape[1]), lambda i: (i, 0)) for a in blocked]
        + [pl.BlockSpec(a.shape, lambda i, _nd=a.ndim: (0,) * _nd) for a in full]
    )
    out = pl.pallas_call(
        body,
        grid=(n // _BN,),
        in_specs=in_specs,
        out_specs=[pl.BlockSpec((_BN, w), lambda i: (i, 0)) for w in out_widths],
        out_shape=[jax.ShapeDtypeStruct((n, w), jnp.float32) for w in out_widths],
    )(*blocked, *full)
    return out


def _b1(da, db, xr, dinv_o, t1c0_o, t1c1_o):
    deg = da[...][:, :1] + db[...][:, :1] + 1.0
    dinv = lax.rsqrt(deg)
    dinv_o[...] = dinv
    xv = xr[...]
    t1c0_o[...] = xv[:, :16] * dinv
    pad = jnp.zeros((xv.shape[0], 14), jnp.float32)
    t1c1_o[...] = jnp.concatenate([xv[:, 16:18] * dinv, pad], axis=1)


def _b2(s0, t0, s1, t1, dv, w1, bb1, w2, t2c0_o, t2c1_o):
    dinv = dv[...]
    u0 = (s0[...] + t0[...]) * dinv
    u1 = ((s1[...] + t1[...]) * dinv)[:, :2]
    out1 = jnp.concatenate([u0, u1], axis=1)
    h1 = jnp.maximum(
        jnp.dot(out1, w1[...], preferred_element_type=jnp.float32) + bb1[...], 0.0)
    g2 = jnp.dot(h1, w2[...], preferred_element_type=jnp.float32)
    t2c0_o[...] = g2[:, :16] * dinv
    t2c1_o[...] = g2[:, 16:] * dinv


def _b3(s0, t0, s1, t1, dv, bb2, w3, t3_o):
    dinv = dv[...]
    u0 = (s0[...] + t0[...]) * dinv
    u1 = (s1[...] + t1[...]) * dinv
    out2 = jnp.concatenate([u0, u1], axis=1)
    h2 = jnp.maximum(out2 + bb2[...], 0.0)
    g3 = jnp.dot(h2, w3[...], preferred_element_type=jnp.float32)
    t3_o[...] = g3 * dinv


def _b4(sa, sb, t3, dv, bb3, w4, t4_o):
    dinv = dv[...]
    out3 = (sa[...] + sb[...] + t3[...]) * dinv
    h3 = jnp.maximum(out3 + bb3[...], 0.0)
    g4 = jnp.dot(h3, w4[...], preferred_element_type=jnp.float32)
    pad = jnp.zeros((g4.shape[0], 14), jnp.float32)
    t4_o[...] = jnp.concatenate([g4 * dinv, pad], axis=1)


def _b5(sa, sb, t4, dv, bb4, y_o):
    v = ((sa[...] + sb[...] + t4[...]) * dv[...])[:, :2] + bb4[...]
    m = jnp.max(v, axis=1, keepdims=True)
    z = v - m
    lse = jnp.log(jnp.sum(jnp.exp(z), axis=1, keepdims=True))
    y_o[...] = z - lse


def kernel(x, edge_index, W1, b1, W2, b2, W3, b3, W4, b4):
    n = x.shape[0]
    e = edge_index.shape[1]
    assert e % _BLK == 0 and n % _BN == 0
    nb = e // _BLK
    ra, _ = _tile_rows(n)

    src2 = edge_index[0].reshape(nb, _BLK)
    dst2 = edge_index[1].reshape(nb, _BLK)
    zeros16 = jnp.zeros((ra, 16), jnp.float32)
    ones16 = jnp.zeros((_BLK, 16), jnp.float32).at[:, 0].set(1.0)
    bb1 = b1.reshape(1, -1)
    bb2 = b2.reshape(1, -1)
    bb3 = b3.reshape(1, -1)
    bb4 = b4.reshape(1, -1)

    prop16 = _make_prop(n, nb, 16)
    prop16c = _make_prop(n, nb, 16, chunked=True)

    degp = _make_deg(n, nb)(dst2, ones16, zeros16)
    dinv, t1c0, t1c1 = _rowcall(_b1, n, [1, 16, 16], [degp[0], degp[1], x], [])

    s1 = prop16c(jnp.stack([t1c0, t1c1]), src2, dst2, zeros16)
    t2c0, t2c1 = _rowcall(
        _b2, n, [16, 16],
        [s1[0], t1c0, s1[1], t1c1, dinv], [W1, bb1, W2])

    s2 = prop16c(jnp.stack([t2c0, t2c1]), src2, dst2, zeros16)
    (t3,) = _rowcall(
        _b3, n, [16],
        [s2[0], t2c0, s2[1], t2c1, dinv], [bb2, W3])

    s3 = prop16(t3, src2, dst2, zeros16)
    (t4,) = _rowcall(_b4, n, [16], [s3[0], s3[1], t3, dinv], [bb3, W4])

    s4 = prop16(t4, src2, dst2, zeros16)
    (y,) = _rowcall(_b5, n, [2], [s4[0], s4[1], t4, dinv], [bb4])
    return y


# speedup vs baseline: 22.2238x; 1.0031x over previous
import functools

import jax
import jax.numpy as jnp
from jax import lax
from jax.experimental import pallas as pl
from jax.experimental.pallas import tpu as pltpu
from jax.experimental.pallas import tpu_sc as plsc

_BLK = 64
_K = 8
_NTILE = 16
_NSC = 2
_BN = 2000


def _tile_rows(n):
    ra = ((n // _NTILE) + 7) // 8 * 8
    rl = n - (_NTILE - 1) * ra
    assert 0 < rl <= ra
    return ra, rl


def _zero_acc_start(sid, zeros, acc, ra, rl, sem):
    base = sid * ra

    @pl.when(sid < _NTILE - 1)
    def _():
        pltpu.async_copy(zeros, acc.at[pl.ds(base, ra)], sem)

    @pl.when(sid == _NTILE - 1)
    def _():
        pltpu.async_copy(zeros.at[pl.ds(0, rl)], acc.at[pl.ds(base, rl)], sem)


def _zero_acc_wait(sid, zeros, acc, ra, rl, sem):
    base = sid * ra

    @pl.when(sid < _NTILE - 1)
    def _():
        pltpu.make_async_copy(zeros, acc.at[pl.ds(base, ra)], sem).wait()

    @pl.when(sid == _NTILE - 1)
    def _():
        pltpu.make_async_copy(zeros.at[pl.ds(0, rl)],
                              acc.at[pl.ds(base, rl)], sem).wait()


def _flush_acc(cid, sid, acc, out, ra, rl):
    base = sid * ra

    @pl.when(sid < _NTILE - 1)
    def _():
        pltpu.sync_copy(acc.at[pl.ds(base, ra)], out.at[cid].at[pl.ds(base, ra)])

    @pl.when(sid == _NTILE - 1)
    def _():
        pltpu.sync_copy(acc.at[pl.ds(base, rl)], out.at[cid].at[pl.ds(base, rl)])


@functools.lru_cache(maxsize=None)
def _make_prop(n, nb, width, chunked=False):
    ra, rl = _tile_rows(n)
    nw = _NTILE if chunked else _NSC * _NTILE
    nsb = nb // _K
    tail = nb - nsb * _K
    mesh = plsc.VectorSubcoreMesh(core_axis_name="c", subcore_axis_name="s",
                                  num_cores=_NSC, num_subcores=_NTILE)
    @functools.partial(
        pl.kernel,
        out_type=jax.ShapeDtypeStruct((_NSC, n, width), jnp.float32),
        mesh=mesh,
        compiler_params=pltpu.CompilerParams(use_tc_tiling_on_sc=False),
        scratch_types=[
            pltpu.VMEM((2, _K, 2, _BLK), jnp.int32),
            pltpu.VMEM((2, _K, _BLK, width), jnp.float32),
            pltpu.VMEM_SHARED((n, width), jnp.float32),
            pltpu.SemaphoreType.DMA,
            pltpu.SemaphoreType.DMA,
            pltpu.SemaphoreType.DMA,
        ],
    )
    def k(table_in, edb, zeros, out, idx, rows, acc,
          gsem, ssem, zsem):
        cid = lax.axis_index("c")
        sid = lax.axis_index("s")
        table = table_in.at[cid] if chunked else table_in
        _zero_acc_start(sid, zeros, acc, ra, rl, zsem)
        w = sid if chunked else sid * _NSC + cid
        cnt = (nsb - w + nw - 1) // nw

        def load_and_fire(b, i):
            blk0 = (w + i * nw) * _K
            pltpu.sync_copy(edb.at[pl.ds(blk0, _K)], idx.at[b])
            for j in range(_K):
                pltpu.async_copy(table.at[idx.at[b].at[j].at[0]],
                                 rows.at[b].at[j], gsem)

        def drain_gathers(b):
            for j in range(_K):
                pltpu.make_async_copy(table.at[idx.at[b].at[j].at[0]],
                                      rows.at[b].at[j], gsem).wait()

        def fire_scatters(b):
            for j in range(_K):
                pltpu.async_copy(rows.at[b].at[j], acc.at[idx.at[b].at[j].at[1]],
                                 ssem, add=True)

        def drain_scatters():
            for j in range(_K):
                pltpu.make_async_copy(rows.at[0].at[j], acc.at[idx.at[0].at[j].at[1]],
                                      ssem).wait()

        def step(i, bx, by):
            @pl.when(i + 1 < cnt)
            def _():
                @pl.when(i >= 1)
                def _():
                    drain_scatters()
                load_and_fire(by, i + 1)
            drain_gathers(bx)
            fire_scatters(bx)

        @pl.when(cnt > 0)
        def _():
            load_and_fire(0, 0)

        _zero_acc_wait(sid, zeros, acc, ra, rl, zsem)
        plsc.subcore_barrier()

        def pair(p, carry):
            step(2 * p, 0, 1)

            @pl.when(2 * p + 1 < cnt)
            def _():
                step(2 * p + 1, 1, 0)
            return carry

        lax.fori_loop(0, (cnt + 1) // 2, pair, 0)
        @pl.when(cnt > 1)
        def _():
            drain_scatters()
        @pl.when(cnt > 0)
        def _():
            drain_scatters()

        if tail:
            @pl.when(w == 0)
            def _():
                blk0 = nsb * _K
                pltpu.sync_copy(edb.at[pl.ds(blk0, tail)],
                                idx.at[0].at[pl.ds(0, tail)])
                for j in range(tail):
                    pltpu.async_copy(table.at[idx.at[0].at[j].at[0]],
                                     rows.at[0].at[j], gsem)
                for j in range(tail):
                    pltpu.make_async_copy(table.at[idx.at[0].at[j].at[0]],
                                          rows.at[0].at[j], gsem).wait()
                for j in range(tail):
                    pltpu.async_copy(rows.at[0].at[j], acc.at[idx.at[0].at[j].at[1]],
                                     ssem, add=True)
                for j in range(tail):
                    pltpu.make_async_copy(rows.at[0].at[j],
                                          acc.at[idx.at[0].at[j].at[1]], ssem).wait()
        plsc.subcore_barrier()
        _flush_acc(cid, sid, acc, out, ra, rl)

    return k


@functools.lru_cache(maxsize=None)
def _make_deg(n, nb):
    width = 16
    ra, rl = _tile_rows(n)
    nw = _NSC * _NTILE
    nsb = nb // _K
    tail = nb - nsb * _K
    mesh = plsc.VectorSubcoreMesh(core_axis_name="c", subcore_axis_name="s",
                                  num_cores=_NSC, num_subcores=_NTILE)

    @functools.partial(
        pl.kernel,
        out_type=jax.ShapeDtypeStruct((_NSC, n, width), jnp.float32),
        mesh=mesh,
        compiler_params=pltpu.CompilerParams(use_tc_tiling_on_sc=False),
        scratch_types=[
            pltpu.VMEM((_K, _BLK), jnp.int32),
            pltpu.VMEM((_BLK, width), jnp.float32),
            pltpu.VMEM_SHARED((n, width), jnp.float32),
            pltpu.SemaphoreType.DMA,
            pltpu.SemaphoreType.DMA,
        ],
    )
    def k(dstb, ones, zeros, out, didx, ones_v, acc, ssem, zsem):
        cid = lax.axis_index("c")
        sid = lax.axis_index("s")
        _zero_acc_start(sid, zeros, acc, ra, rl, zsem)
        pltpu.sync_copy(ones, ones_v)
        _zero_acc_wait(sid, zeros, acc, ra, rl, zsem)
        plsc.subcore_barrier()
        w = sid * _NSC + cid
        cnt = (nsb - w + nw - 1) // nw

        def run_blocks(blk0, m):
            pltpu.sync_copy(dstb.at[pl.ds(blk0, m)], didx.at[pl.ds(0, m)])
            for j in range(m):
                pltpu.async_copy(ones_v, acc.at[didx.at[j]], ssem, add=True)
            for j in range(m):
                pltpu.make_async_copy(ones_v, acc.at[didx.at[j]], ssem).wait()

        def body(i, carry):
            run_blocks((w + i * nw) * _K, _K)
            return carry

        lax.fori_loop(0, cnt, body, 0)
        if tail:
            @pl.when(w == 0)
            def _():
                run_blocks(nsb * _K, tail)
        plsc.subcore_barrier()
        _flush_acc(cid, sid, acc, out, ra, rl)

    return k


def _rowcall(body, n, out_widths, blocked, full):
    in_specs = (
        [pl.BlockSpec((_BN, a.shape[1]), lambda i: (i, 0)) for a in blocked]
        + [pl.BlockSpec(a.shape, lambda i, _nd=a.ndim: (0,) * _nd) for a in full]
    )
    out = pl.pallas_call(
        body,
        grid=(n // _BN,),
        in_specs=in_specs,
        out_specs=[pl.BlockSpec((_BN, w), lambda i: (i, 0)) for w in out_widths],
        out_shape=[jax.ShapeDtypeStruct((n, w), jnp.float32) for w in out_widths],
    )(*blocked, *full)
    return out


def _b1(da, db, xr, dinv_o, t1c0_o, t1c1_o):
    deg = da[...][:, :1] + db[...][:, :1] + 1.0
    dinv = lax.rsqrt(deg)
    dinv_o[...] = dinv
    xv = xr[...]
    t1c0_o[...] = xv[:, :16] * dinv
    pad = jnp.zeros((xv.shape[0], 14), jnp.float32)
    t1c1_o[...] = jnp.concatenate([xv[:, 16:18] * dinv, pad], axis=1)


def _b2(s0, t0, s1, t1, dv, w1, bb1, w2, t2c0_o, t2c1_o):
    dinv = dv[...]
    u0 = (s0[...] + t0[...]) * dinv
    u1 = ((s1[...] + t1[...]) * dinv)[:, :2]
    out1 = jnp.concatenate([u0, u1], axis=1)
    h1 = jnp.maximum(
        jnp.dot(out1, w1[...], preferred_element_type=jnp.float32) + bb1[...], 0.0)
    g2 = jnp.dot(h1, w2[...], preferred_element_type=jnp.float32)
    t2c0_o[...] = g2[:, :16] * dinv
    t2c1_o[...] = g2[:, 16:] * dinv


def _b3(s0, t0, s1, t1, dv, bb2, w3, t3_o):
    dinv = dv[...]
    u0 = (s0[...] + t0[...]) * dinv
    u1 = (s1[...] + t1[...]) * dinv
    out2 = jnp.concatenate([u0, u1], axis=1)
    h2 = jnp.maximum(out2 + bb2[...], 0.0)
    g3 = jnp.dot(h2, w3[...], preferred_element_type=jnp.float32)
    t3_o[...] = g3 * dinv


def _b4(sa, sb, t3, dv, bb3, w4, t4_o):
    dinv = dv[...]
    out3 = (sa[...] + sb[...] + t3[...]) * dinv
    h3 = jnp.maximum(out3 + bb3[...], 0.0)
    g4 = jnp.dot(h3, w4[...], preferred_element_type=jnp.float32)
    pad = jnp.zeros((g4.shape[0], 14), jnp.float32)
    t4_o[...] = jnp.concatenate([g4 * dinv, pad], axis=1)


def _b5(sa, sb, t4, dv, bb4, y_o):
    v = ((sa[...] + sb[...] + t4[...]) * dv[...])[:, :2] + bb4[...]
    m = jnp.max(v, axis=1, keepdims=True)
    z = v - m
    lse = jnp.log(jnp.sum(jnp.exp(z), axis=1, keepdims=True))
    y_o[...] = z - lse


def kernel(x, edge_index, W1, b1, W2, b2, W3, b3, W4, b4):
    n = x.shape[0]
    e = edge_index.shape[1]
    assert e % _BLK == 0 and n % _BN == 0
    nb = e // _BLK
    ra, _ = _tile_rows(n)

    src2 = edge_index[0].reshape(nb, _BLK)
    dst2 = edge_index[1].reshape(nb, _BLK)
    ed2 = jnp.stack([src2, dst2], axis=1)
    zeros16 = jnp.zeros((ra, 16), jnp.float32)
    ones16 = jnp.zeros((_BLK, 16), jnp.float32).at[:, 0].set(1.0)
    bb1 = b1.reshape(1, -1)
    bb2 = b2.reshape(1, -1)
    bb3 = b3.reshape(1, -1)
    bb4 = b4.reshape(1, -1)

    prop16 = _make_prop(n, nb, 16)
    prop16c = _make_prop(n, nb, 16, chunked=True)

    degp = _make_deg(n, nb)(dst2, ones16, zeros16)
    dinv, t1c0, t1c1 = _rowcall(_b1, n, [1, 16, 16], [degp[0], degp[1], x], [])

    s1 = prop16c(jnp.stack([t1c0, t1c1]), ed2, zeros16)
    t2c0, t2c1 = _rowcall(
        _b2, n, [16, 16],
        [s1[0], t1c0, s1[1], t1c1, dinv], [W1, bb1, W2])

    s2 = prop16c(jnp.stack([t2c0, t2c1]), ed2, zeros16)
    (t3,) = _rowcall(
        _b3, n, [16],
        [s2[0], t2c0, s2[1], t2c1, dinv], [bb2, W3])

    s3 = prop16(t3, ed2, zeros16)
    (t4,) = _rowcall(_b4, n, [16], [s3[0], s3[1], t3, dinv], [bb3, W4])

    s4 = prop16(t4, ed2, zeros16)
    (y,) = _rowcall(_b5, n, [2], [s4[0], s4[1], t4, dinv], [bb4])
    return y
